# Initial kernel scaffold; baseline (speedup 1.0000x reference)
import functools

import jax
import jax.numpy as jnp
from jax import lax
from jax.experimental import pallas as pl
from jax.experimental.pallas import tpu as pltpu
from jax.experimental.pallas import tpu_sc as plsc

VOCAB = 32
HIDDEN = 128
BATCH = 16384
SEQ = 200

N_TOKENS = BATCH * SEQ
NUM_WORKERS = 32
TOK_PER_WORKER = N_TOKENS // NUM_WORKERS

CHUNK = 512
SUB = 128
N_SUB = CHUNK // SUB
N_CHUNKS = TOK_PER_WORKER // CHUNK

_mesh = plsc.VectorSubcoreMesh(core_axis_name="c", subcore_axis_name="s")


@functools.partial(
    pl.kernel,
    mesh=_mesh,
    out_type=jax.ShapeDtypeStruct((N_TOKENS, HIDDEN), jnp.float32),
    scratch_types=[
        pltpu.VMEM((N_SUB, SUB), jnp.int32),
        pltpu.VMEM((CHUNK, HIDDEN), jnp.float32),
        pltpu.SemaphoreType.DMA,
    ],
)
def _embed_lookup(ids_hbm, table_hbm, out_hbm, idx_v, rows_v, sem):
    wid = lax.axis_index("s") * 2 + lax.axis_index("c")
    base = wid * TOK_PER_WORKER

    def body(c, carry):
        off = base + c * CHUNK
        pltpu.sync_copy(ids_hbm.at[pl.ds(off, CHUNK)], idx_v)
        copies = []
        for j in range(N_SUB):
            copies.append(
                pltpu.async_copy(
                    table_hbm.at[idx_v.at[j]],
                    rows_v.at[pl.ds(j * SUB, SUB)],
                    sem,
                )
            )
        for cp in copies:
            cp.wait()
        pltpu.sync_copy(rows_v, out_hbm.at[pl.ds(off, CHUNK)])
        return carry

    lax.fori_loop(0, N_CHUNKS, body, 0)


def kernel(input_ids, attention_mask, embedding_table):
    del attention_mask
    ids = input_ids.reshape(N_TOKENS).astype(jnp.int32)
    out = _embed_lookup(ids, embedding_table)
    return out.reshape(BATCH, SEQ, HIDDEN)


# speedup vs baseline: 2.2430x; 2.2430x over previous
import functools

import jax
import jax.numpy as jnp
from jax import lax
from jax.experimental import pallas as pl
from jax.experimental.pallas import tpu as pltpu
from jax.experimental.pallas import tpu_sc as plsc

VOCAB = 32
HIDDEN = 128
BATCH = 16384
SEQ = 200

N_TOKENS = BATCH * SEQ
NUM_WORKERS = 32
TOK_PER_WORKER = N_TOKENS // NUM_WORKERS

CHUNK = 400
N_CHUNKS = TOK_PER_WORKER // CHUNK
GROUPS = CHUNK // 16
CHUNK_ELEMS = CHUNK * HIDDEN

_mesh = plsc.VectorSubcoreMesh(core_axis_name="c", subcore_axis_name="s")


@functools.partial(
    pl.kernel,
    mesh=_mesh,
    compiler_params=pltpu.CompilerParams(needs_layout_passes=False),
    out_type=jax.ShapeDtypeStruct((N_TOKENS * HIDDEN,), jnp.float32),
    scratch_types=[
        pltpu.VMEM((VOCAB * HIDDEN,), jnp.float32),
        pltpu.VMEM((CHUNK,), jnp.int32),
        pltpu.VMEM((CHUNK,), jnp.int32),
        pltpu.VMEM((CHUNK_ELEMS,), jnp.float32),
        pltpu.VMEM((CHUNK_ELEMS,), jnp.float32),
        pltpu.SemaphoreType.DMA,
        pltpu.SemaphoreType.DMA,
        pltpu.SemaphoreType.DMA,
        pltpu.SemaphoreType.DMA,
    ],
)
def _embed_lookup(ids_hbm, table_hbm, out_hbm, table_v, ids0, ids1,
                  obuf0, obuf1, si0, si1, so0, so1):
    wid = lax.axis_index("s") * 2 + lax.axis_index("c")
    base = wid * TOK_PER_WORKER
    idsv = (ids0, ids1)
    obuf = (obuf0, obuf1)
    sem_i = (si0, si1)
    sem_o = (so0, so1)

    pltpu.sync_copy(table_hbm, table_v)

    lane = lax.iota(jnp.int32, 16)
    lane_off = lane * HIDDEN

    pltpu.async_copy(ids_hbm.at[pl.ds(base, CHUNK)], idsv[0], sem_i[0])

    def expand(ids_ref, buf_ref):
        def g_body(g, carry):
            ids16 = ids_ref[pl.ds(g * 16, 16)]
            src_base = ids16 * HIDDEN
            dst_base = lane_off + g * (16 * HIDDEN)

            @plsc.parallel_loop(0, HIDDEN, unroll=8)
            def j_body(j):
                v = plsc.load_gather(table_v, [src_base + j])
                plsc.store_scatter(buf_ref, [dst_base + j], v)

            return carry
        lax.fori_loop(0, GROUPS, g_body, 0, unroll=False)

    def super_body(s, carry):
        for b in range(2):
            c = s * 2 + b
            off = base + c * CHUNK
            pltpu.make_async_copy(
                ids_hbm.at[pl.ds(off, CHUNK)], idsv[b], sem_i[b]).wait()
            if b == 0:
                pltpu.async_copy(
                    ids_hbm.at[pl.ds(off + CHUNK, CHUNK)], idsv[1], sem_i[1])
            else:
                @pl.when(s < (N_CHUNKS // 2) - 1)
                def _():
                    pltpu.async_copy(
                        ids_hbm.at[pl.ds(off + CHUNK, CHUNK)],
                        idsv[0], sem_i[0])
            @pl.when(s > 0)
            def _():
                pltpu.make_async_copy(
                    obuf[b], out_hbm.at[pl.ds(off * HIDDEN, CHUNK_ELEMS)],
                    sem_o[b]).wait()
            expand(idsv[b], obuf[b])
            pltpu.async_copy(
                obuf[b], out_hbm.at[pl.ds(off * HIDDEN, CHUNK_ELEMS)],
                sem_o[b])
        return carry

    lax.fori_loop(0, N_CHUNKS // 2, super_body, 0, unroll=False)

    for b in range(2):
        off = base + (N_CHUNKS - 2 + b) * CHUNK
        pltpu.make_async_copy(
            obuf[b], out_hbm.at[pl.ds(off * HIDDEN, CHUNK_ELEMS)],
            sem_o[b]).wait()


def kernel(input_ids, attention_mask, embedding_table):
    del attention_mask
    ids = input_ids.reshape(N_TOKENS).astype(jnp.int32)
    table = embedding_table.reshape(VOCAB * HIDDEN)
    out = _embed_lookup(ids, table)
    return out.reshape(BATCH, SEQ, HIDDEN)


# speedup vs baseline: 14.9937x; 6.6848x over previous
import functools

import jax
import jax.numpy as jnp
from jax import lax
from jax.experimental import pallas as pl
from jax.experimental.pallas import tpu as pltpu
from jax.experimental.pallas import tpu_sc as plsc

VOCAB = 32
HIDDEN = 128
BATCH = 16384
SEQ = 200

N_TOKENS = BATCH * SEQ
NUM_WORKERS = 32
TOK_PER_WORKER = N_TOKENS // NUM_WORKERS

CHUNK = 400
N_CHUNKS = TOK_PER_WORKER // CHUNK
GROUPS = CHUNK // 16
CHUNK_ELEMS = CHUNK * HIDDEN

_mesh = plsc.VectorSubcoreMesh(core_axis_name="c", subcore_axis_name="s")


@functools.partial(
    pl.kernel,
    mesh=_mesh,
    compiler_params=pltpu.CompilerParams(needs_layout_passes=False),
    out_type=jax.ShapeDtypeStruct((N_TOKENS * HIDDEN,), jnp.float32),
    scratch_types=[
        pltpu.VMEM((VOCAB * HIDDEN,), jnp.float32),
        pltpu.VMEM((CHUNK,), jnp.int32),
        pltpu.VMEM((CHUNK,), jnp.int32),
        pltpu.VMEM((CHUNK_ELEMS,), jnp.float32),
        pltpu.VMEM((CHUNK_ELEMS,), jnp.float32),
        pltpu.SemaphoreType.DMA,
        pltpu.SemaphoreType.DMA,
        pltpu.SemaphoreType.DMA,
        pltpu.SemaphoreType.DMA,
    ],
)
def _embed_lookup(ids_hbm, table_hbm, out_hbm, table_v, ids0, ids1,
                  obuf0, obuf1, si0, si1, so0, so1):
    wid = lax.axis_index("s") * 2 + lax.axis_index("c")
    base = wid * TOK_PER_WORKER
    idsv = (ids0, ids1)
    obuf = (obuf0, obuf1)
    sem_i = (si0, si1)
    sem_o = (so0, so1)

    pltpu.sync_copy(table_hbm, table_v)

    lane = lax.iota(jnp.int32, 16)
    lane_off = lane * HIDDEN

    pltpu.async_copy(ids_hbm.at[pl.ds(base, CHUNK)], idsv[0], sem_i[0])

    def expand(ids_ref, buf_ref):
        @plsc.parallel_loop(0, GROUPS, unroll=1)
        def g_body(g):
            v16 = ids_ref[pl.ds(g * 16, 16)]
            dst_g = g * (16 * HIDDEN)
            for k in range(16):
                src0 = v16[k] * HIDDEN
                dst0 = dst_g + k * HIDDEN
                for jb in range(0, HIDDEN, 16):
                    buf_ref[pl.ds(dst0 + jb, 16)] = (
                        table_v[pl.ds(src0 + jb, 16)])

    def super_body(s, carry):
        for b in range(2):
            c = s * 2 + b
            off = base + c * CHUNK
            pltpu.make_async_copy(
                ids_hbm.at[pl.ds(off, CHUNK)], idsv[b], sem_i[b]).wait()
            if b == 0:
                pltpu.async_copy(
                    ids_hbm.at[pl.ds(off + CHUNK, CHUNK)], idsv[1], sem_i[1])
            else:
                @pl.when(s < (N_CHUNKS // 2) - 1)
                def _():
                    pltpu.async_copy(
                        ids_hbm.at[pl.ds(off + CHUNK, CHUNK)],
                        idsv[0], sem_i[0])
            @pl.when(s > 0)
            def _():
                pltpu.make_async_copy(
                    obuf[b], out_hbm.at[pl.ds(off * HIDDEN, CHUNK_ELEMS)],
                    sem_o[b]).wait()
            expand(idsv[b], obuf[b])
            pltpu.async_copy(
                obuf[b], out_hbm.at[pl.ds(off * HIDDEN, CHUNK_ELEMS)],
                sem_o[b])
        return carry

    lax.fori_loop(0, N_CHUNKS // 2, super_body, 0, unroll=False)

    for b in range(2):
        off = base + (N_CHUNKS - 2 + b) * CHUNK
        pltpu.make_async_copy(
            obuf[b], out_hbm.at[pl.ds(off * HIDDEN, CHUNK_ELEMS)],
            sem_o[b]).wait()


def kernel(input_ids, attention_mask, embedding_table):
    del attention_mask
    ids = input_ids.reshape(N_TOKENS).astype(jnp.int32)
    table = embedding_table.reshape(VOCAB * HIDDEN)
    out = _embed_lookup(ids, table)
    return out.reshape(BATCH, SEQ, HIDDEN)
